# baseline (speedup 1.0000x reference)
import functools
import jax
import jax.numpy as jnp
from jax import lax
from jax.experimental import pallas as pl
from jax.experimental.pallas import tpu as pltpu
from jax.experimental.pallas import tpu_sc as plsc

N = 50000
E = 800000
NG = 256
HID = 64
EDGE = 16
L = 3
NS = 4
MAX_Z = 118

NP_ = 50176
NBLK = 1792
NGRID = 28
EBLK = 4000
EGRID = E // EBLK

_f32 = jnp.float32


def _softplus(x):
    u = jnp.exp(-jnp.abs(x))
    zz = u / (2.0 + u)
    z2 = zz * zz
    p = 1.0 + z2 * (1.0 / 3.0 + z2 * (0.2 + z2 * (1.0 / 7.0)))
    return jnp.maximum(x, 0.0) + 2.0 * zz * p


def _x0_body(zc_ref, xs_ref, ae_ref, waet_ref, b0_ref, wxs_ref, wdt_ref, wst_ref,
             x0_ref, afs_ref, bfs_ref):
    zc = zc_ref[0, 0, :]
    t = jnp.dot(ae_ref[...], waet_ref[...], preferred_element_type=_f32, precision=lax.Precision.HIGHEST)
    oh = (zc[:, None] == lax.broadcasted_iota(jnp.int32, (NBLK, 128), 1)).astype(_f32)
    g = jnp.dot(oh, t, preferred_element_type=_f32, precision=lax.Precision.HIGHEST)
    x0 = g + jnp.dot(xs_ref[...], wxs_ref[...], preferred_element_type=_f32, precision=lax.Precision.HIGHEST) + b0_ref[...]
    x0_ref[...] = x0
    afs_ref[...] = jnp.dot(x0, wdt_ref[...], preferred_element_type=_f32, precision=lax.Precision.HIGHEST)
    bfs_ref[...] = jnp.dot(x0, wst_ref[...], preferred_element_type=_f32, precision=lax.Precision.HIGHEST)


def _x0_call(zc3, xs_p, ae_pad, waet, b0, wxs, wdt, wst):
    return pl.pallas_call(
        _x0_body,
        grid=(NGRID,),
        in_specs=[
            pl.BlockSpec((1, 1, NBLK), lambda i: (i, 0, 0)),
            pl.BlockSpec((NBLK, 8), lambda i: (i, 0)),
            pl.BlockSpec((128, 64), lambda i: (0, 0)),
            pl.BlockSpec((64, 64), lambda i: (0, 0)),
            pl.BlockSpec((1, 64), lambda i: (0, 0)),
            pl.BlockSpec((8, 64), lambda i: (0, 0)),
            pl.BlockSpec((64, 128), lambda i: (0, 0)),
            pl.BlockSpec((64, 128), lambda i: (0, 0)),
        ],
        out_specs=[
            pl.BlockSpec((NBLK, 64), lambda i: (i, 0)),
            pl.BlockSpec((NBLK, 128), lambda i: (i, 0)),
            pl.BlockSpec((NBLK, 128), lambda i: (i, 0)),
        ],
        out_shape=[
            jax.ShapeDtypeStruct((NP_, 64), _f32),
            jax.ShapeDtypeStruct((NP_, 128), _f32),
            jax.ShapeDtypeStruct((NP_, 128), _f32),
        ],
    )(zc3, xs_p, ae_pad, waet, b0, wxs, wdt, wst)


def _efs_body(ea_ref, w_ref, o_ref):
    o_ref[...] = jnp.dot(ea_ref[...], w_ref[...], preferred_element_type=_f32, precision=lax.Precision.HIGHEST)


def _efs_call(ea_aug, w_aug):
    return pl.pallas_call(
        _efs_body,
        grid=(EGRID,),
        in_specs=[
            pl.BlockSpec((EBLK, 24), lambda i: (i, 0)),
            pl.BlockSpec((24, 128), lambda i: (0, 0)),
        ],
        out_specs=pl.BlockSpec((EBLK, 128), lambda i: (i, 0)),
        out_shape=jax.ShapeDtypeStruct((E, 128), _f32),
    )(ea_aug, w_aug)


def _ln_relu(x, agg, g_ref, b_ref):
    y = x + agg
    mu = jnp.mean(y, axis=1, keepdims=True)
    d = y - mu
    var = jnp.mean(d * d, axis=1, keepdims=True)
    xn = d * lax.rsqrt(var + 1e-5) * g_ref[...] + b_ref[...]
    return jnp.maximum(xn, 0.0)


def _post_xform_body(x_ref, agg_ref, g_ref, b_ref, wdt_ref, wst_ref,
                     xn_ref, afs_ref, bfs_ref):
    xn = _ln_relu(x_ref[...], agg_ref[...], g_ref, b_ref)
    xn_ref[...] = xn
    afs_ref[...] = jnp.dot(xn, wdt_ref[...], preferred_element_type=_f32, precision=lax.Precision.HIGHEST)
    bfs_ref[...] = jnp.dot(xn, wst_ref[...], preferred_element_type=_f32, precision=lax.Precision.HIGHEST)


def _post_xform_call(x, agg, g, b, wdt, wst):
    return pl.pallas_call(
        _post_xform_body,
        grid=(NGRID,),
        in_specs=[
            pl.BlockSpec((NBLK, 64), lambda i: (i, 0)),
            pl.BlockSpec((NBLK, 64), lambda i: (i, 0)),
            pl.BlockSpec((1, 64), lambda i: (0, 0)),
            pl.BlockSpec((1, 64), lambda i: (0, 0)),
            pl.BlockSpec((64, 128), lambda i: (0, 0)),
            pl.BlockSpec((64, 128), lambda i: (0, 0)),
        ],
        out_specs=[
            pl.BlockSpec((NBLK, 64), lambda i: (i, 0)),
            pl.BlockSpec((NBLK, 128), lambda i: (i, 0)),
            pl.BlockSpec((NBLK, 128), lambda i: (i, 0)),
        ],
        out_shape=[
            jax.ShapeDtypeStruct((NP_, 64), _f32),
            jax.ShapeDtypeStruct((NP_, 128), _f32),
            jax.ShapeDtypeStruct((NP_, 128), _f32),
        ],
    )(x, agg, g, b, wdt, wst)


def _post_body(x_ref, agg_ref, g_ref, b_ref, xn_ref):
    xn_ref[...] = _ln_relu(x_ref[...], agg_ref[...], g_ref, b_ref)


def _post_call(x, agg, g, b):
    return pl.pallas_call(
        _post_body,
        grid=(NGRID,),
        in_specs=[
            pl.BlockSpec((NBLK, 64), lambda i: (i, 0)),
            pl.BlockSpec((NBLK, 64), lambda i: (i, 0)),
            pl.BlockSpec((1, 64), lambda i: (0, 0)),
            pl.BlockSpec((1, 64), lambda i: (0, 0)),
        ],
        out_specs=pl.BlockSpec((NBLK, 64), lambda i: (i, 0)),
        out_shape=jax.ShapeDtypeStruct((NP_, 64), _f32),
    )(x, agg, g, b)


def _pool_body(x_ref, b3_ref, w1t_ref, b1_ref, w2t_ref, b2_ref, o_ref, acc_ref):
    i = pl.program_id(0)

    @pl.when(i == 0)
    def _():
        acc_ref[...] = jnp.zeros_like(acc_ref)

    bb = b3_ref[0, 0, :]
    oh = (bb[:, None] == lax.broadcasted_iota(jnp.int32, (NBLK, NG), 1)).astype(_f32)
    xa = jnp.concatenate([x_ref[...], jnp.ones((NBLK, 64), _f32)], axis=1)
    acc_ref[...] += lax.dot_general(oh, xa, (((0,), (0,)), ((), ())),
                                    preferred_element_type=_f32, precision=lax.Precision.HIGHEST)

    @pl.when(i == NGRID - 1)
    def _():
        s = acc_ref[:, :64]
        c = acc_ref[:, 64:65]
        pooled = s / jnp.maximum(c, 1.0)
        h = jnp.maximum(jnp.dot(pooled, w1t_ref[...], preferred_element_type=_f32, precision=lax.Precision.HIGHEST)
                        + b1_ref[...], 0.0)
        o_ref[...] = jnp.dot(h, w2t_ref[...], preferred_element_type=_f32, precision=lax.Precision.HIGHEST) + b2_ref[...]


def _pool_call(x, batch3, w1t, b1, w2t, b2):
    return pl.pallas_call(
        _pool_body,
        grid=(NGRID,),
        in_specs=[
            pl.BlockSpec((NBLK, 64), lambda i: (i, 0)),
            pl.BlockSpec((1, 1, NBLK), lambda i: (i, 0, 0)),
            pl.BlockSpec((64, 32), lambda i: (0, 0)),
            pl.BlockSpec((1, 32), lambda i: (0, 0)),
            pl.BlockSpec((32, 8), lambda i: (0, 0)),
            pl.BlockSpec((1, 8), lambda i: (0, 0)),
        ],
        out_specs=pl.BlockSpec((NG, 8), lambda i: (0, 0)),
        out_shape=jax.ShapeDtypeStruct((NG, 8), _f32),
        scratch_shapes=[pltpu.VMEM((NG, 128), _f32)],
    )(x, batch3, w1t, b1, w2t, b2)


def _edge_stage(afs, bfs, efs, src, dst):
    gp = afs[dst] + bfs[src] + efs
    gate = jax.nn.sigmoid(gp[:, :64])
    core = _softplus(gp[:, 64:])
    return jax.ops.segment_sum(gate * core, dst, num_segments=NP_)


NC_SC = 2
NS_SC = 16
NW = NC_SC * NS_SC
CCH = 224
NCHUNK = NP_ // CCH
CPT = NCHUNK // NW
EB = 128


def _edge_sc(dst_s, src_s, eid, eoff_pad, afs, bfs, efs, zrows):
    mesh = plsc.VectorSubcoreMesh(core_axis_name="c", subcore_axis_name="s")

    @functools.partial(
        pl.kernel,
        out_type=jax.ShapeDtypeStruct((NP_, 64), _f32),
        mesh=mesh,
        compiler_params=pltpu.CompilerParams(needs_layout_passes=False),
        scratch_types=[
            pltpu.VMEM((16, 16), jnp.int32),
            pltpu.VMEM((EB,), jnp.int32),
            pltpu.VMEM((EB,), jnp.int32),
            pltpu.VMEM((EB,), jnp.int32),
            pltpu.VMEM((CCH, 128), _f32),
            pltpu.VMEM((EB, 128), _f32),
            pltpu.VMEM((EB, 128), _f32),
            pltpu.VMEM((CCH, 64), _f32),
            pltpu.SemaphoreType.DMA,
            pltpu.SemaphoreType.DMA,
        ],
    )
    def k(dst_hbm, src_hbm, eid_hbm, eoff_hbm, afs_hbm, bfs_hbm, efs_hbm, z_hbm,
          agg_hbm, eoff_v, srcb, eidb, dstb, afsb, bfsb, efsb, accb, sem1, sem2):
        wid = lax.axis_index("s") * NC_SC + lax.axis_index("c")
        pltpu.sync_copy(eoff_hbm, eoff_v)
        iota16 = lax.iota(jnp.int32, 16)

        def chunk_body(j, carry):
            c = wid * CPT + j
            cn0 = c * CCH
            c16 = jnp.full((16,), 0, jnp.int32) + c
            c16b = c16 + 1
            e_lo = jnp.max(plsc.load_gather(eoff_v, [c16 >> 4, c16 & 15]))
            e_hi = jnp.max(plsc.load_gather(eoff_v, [c16b >> 4, c16b & 15]))
            b0 = e_lo // EB
            b1 = (e_hi + (EB - 1)) // EB
            pltpu.sync_copy(z_hbm, accb)
            pltpu.sync_copy(afs_hbm.at[pl.ds(cn0, CCH)], afsb)

            def batch_body(bi, carry2):
                e0 = bi * EB
                pltpu.sync_copy(src_hbm.at[pl.ds(e0, EB)], srcb)
                pltpu.sync_copy(eid_hbm.at[pl.ds(e0, EB)], eidb)
                pltpu.sync_copy(dst_hbm.at[pl.ds(e0, EB)], dstb)
                cp1 = pltpu.async_copy(bfs_hbm.at[srcb], bfsb, sem1)
                cp2 = pltpu.async_copy(efs_hbm.at[eidb], efsb, sem2)
                cp1.wait()
                cp2.wait()

                def group_body(g, carry3):
                    ei = iota16 + g * 16
                    d16 = plsc.load_gather(dstb, [ei])
                    dloc = d16 - cn0
                    valid = (dloc >= 0) & (dloc < CCH)
                    arow = jnp.clip(dloc, 0, CCH - 1)

                    def feat_body(kf, carry4):
                        cf = jnp.broadcast_to(kf, (16,))
                        cs = cf + 64
                        af = plsc.load_gather(afsb, [arow, cf])
                        bf = plsc.load_gather(bfsb, [ei, cf])
                        ef = plsc.load_gather(efsb, [ei, cf])
                        as_ = plsc.load_gather(afsb, [arow, cs])
                        bs = plsc.load_gather(bfsb, [ei, cs])
                        es = plsc.load_gather(efsb, [ei, cs])
                        tf = af + bf + ef
                        ts = as_ + bs + es
                        gate = 1.0 / (1.0 + jnp.exp(-tf))
                        u = jnp.exp(-jnp.abs(ts))
                        zz = u / (2.0 + u)
                        z2 = zz * zz
                        p = 1.0 + z2 * (1.0 / 3.0 + z2 * (0.2 + z2 * (1.0 / 7.0)))
                        sp = jnp.maximum(ts, 0.0) + 2.0 * zz * p
                        plsc.addupdate_scatter(accb, [arow, cf], gate * sp, mask=valid)
                        return carry4

                    return lax.fori_loop(0, 64, feat_body, carry3)

                return lax.fori_loop(0, EB // 16, group_body, carry2)

            lax.fori_loop(b0, b1, batch_body, 0)
            pltpu.sync_copy(accb, agg_hbm.at[pl.ds(cn0, CCH)])
            return carry

        lax.fori_loop(0, CPT, chunk_body, 0)

    return k(dst_s, src_s, eid, eoff_pad, afs, bfs, efs, zrows)


def kernel(z, x_scalar, edge_index, edge_attr, batch, atom_embed, lin0_w, lin0_b,
           convf_w, convf_b, convs_w, convs_b, ln_g, ln_b, lin1_w, lin1_b,
           lin2_w, lin2_b):
    src = edge_index[0]
    dst = edge_index[1]
    zc = jnp.clip(z, 0, MAX_Z)

    zc3 = jnp.pad(zc, (0, NP_ - N)).reshape(NGRID, 1, NBLK)
    xs_p = jnp.pad(x_scalar, ((0, NP_ - N), (0, 8 - NS)))
    ae_pad = jnp.pad(atom_embed, ((0, 128 - (MAX_Z + 2)), (0, 0)))
    waet = lin0_w[:, :HID].T
    wxs = jnp.pad(lin0_w[:, HID:].T, ((0, 8 - NS), (0, 0)))
    b0 = lin0_b[None, :]

    wdt = [jnp.concatenate([convf_w[l][:, :HID].T, convs_w[l][:, :HID].T], axis=1)
           for l in range(L)]
    wst = [jnp.concatenate([convf_w[l][:, HID:2 * HID].T,
                            convs_w[l][:, HID:2 * HID].T], axis=1) for l in range(L)]
    ea_aug = jnp.concatenate(
        [edge_attr, jnp.ones((E, 1), _f32), jnp.zeros((E, 7), _f32)], axis=1)
    we_aug = [jnp.concatenate([
        jnp.concatenate([convf_w[l][:, 2 * HID:].T, convs_w[l][:, 2 * HID:].T], axis=1),
        jnp.concatenate([convf_b[l], convs_b[l]])[None, :],
        jnp.zeros((7, 128), _f32)], axis=0) for l in range(L)]

    dst_s, src_s, eid = lax.sort(
        (dst, src, jnp.arange(E, dtype=jnp.int32)), num_keys=1)
    bounds = jnp.arange(NCHUNK + 1, dtype=jnp.int32) * CCH
    eoff = jnp.searchsorted(dst_s, bounds, side='left').astype(jnp.int32)
    eoff_pad = jnp.pad(eoff, (0, 256 - (NCHUNK + 1)), constant_values=E).reshape(16, 16)
    zrows = jnp.zeros((CCH, 64), _f32)

    x, afs, bfs = _x0_call(zc3, xs_p, ae_pad, waet, b0, wxs, wdt[0], wst[0])
    efs = [_efs_call(ea_aug, we_aug[l]) for l in range(L)]

    for l in range(L):
        agg = _edge_sc(dst_s, src_s, eid, eoff_pad, afs, bfs, efs[l], zrows)
        if l < L - 1:
            x, afs, bfs = _post_xform_call(x, agg, ln_g[l][None, :], ln_b[l][None, :],
                                           wdt[l + 1], wst[l + 1])
        else:
            x = _post_call(x, agg, ln_g[l][None, :], ln_b[l][None, :])

    batch3 = jnp.pad(batch, (0, NP_ - N), constant_values=NG).reshape(NGRID, 1, NBLK)
    w1t = lin1_w.T
    b1 = lin1_b[None, :]
    w2t = jnp.pad(lin2_w.T, ((0, 0), (0, 7)))
    b2 = jnp.pad(lin2_b[None, :], ((0, 0), (0, 7)))
    out2 = _pool_call(x, batch3, w1t, b1, w2t, b2)
    return out2[:, 0]


# speedup vs baseline: 1.0643x; 1.0643x over previous
import functools
import jax
import jax.numpy as jnp
from jax import lax
from jax.experimental import pallas as pl
from jax.experimental.pallas import tpu as pltpu
from jax.experimental.pallas import tpu_sc as plsc

N = 50000
E = 800000
NG = 256
HID = 64
EDGE = 16
L = 3
NS = 4
MAX_Z = 118

NP_ = 50176
NBLK = 1792
NGRID = 28
EBLK = 4000
EGRID = E // EBLK

_f32 = jnp.float32


def _softplus(x):
    u = jnp.exp(-jnp.abs(x))
    zz = u / (2.0 + u)
    z2 = zz * zz
    p = 1.0 + z2 * (1.0 / 3.0 + z2 * (0.2 + z2 * (1.0 / 7.0)))
    return jnp.maximum(x, 0.0) + 2.0 * zz * p


def _x0_body(zc_ref, xs_ref, ae_ref, waet_ref, b0_ref, wxs_ref, wdt_ref, wst_ref,
             x0_ref, afs_ref, bfs_ref):
    zc = zc_ref[0, 0, :]
    t = jnp.dot(ae_ref[...], waet_ref[...], preferred_element_type=_f32, precision=lax.Precision.HIGHEST)
    oh = (zc[:, None] == lax.broadcasted_iota(jnp.int32, (NBLK, 128), 1)).astype(_f32)
    g = jnp.dot(oh, t, preferred_element_type=_f32, precision=lax.Precision.HIGHEST)
    x0 = g + jnp.dot(xs_ref[...], wxs_ref[...], preferred_element_type=_f32, precision=lax.Precision.HIGHEST) + b0_ref[...]
    x0_ref[...] = x0
    afs_ref[...] = jnp.dot(x0, wdt_ref[...], preferred_element_type=_f32, precision=lax.Precision.HIGHEST)
    bfs_ref[...] = jnp.dot(x0, wst_ref[...], preferred_element_type=_f32, precision=lax.Precision.HIGHEST)


def _x0_call(zc3, xs_p, ae_pad, waet, b0, wxs, wdt, wst):
    return pl.pallas_call(
        _x0_body,
        grid=(NGRID,),
        in_specs=[
            pl.BlockSpec((1, 1, NBLK), lambda i: (i, 0, 0)),
            pl.BlockSpec((NBLK, 8), lambda i: (i, 0)),
            pl.BlockSpec((128, 64), lambda i: (0, 0)),
            pl.BlockSpec((64, 64), lambda i: (0, 0)),
            pl.BlockSpec((1, 64), lambda i: (0, 0)),
            pl.BlockSpec((8, 64), lambda i: (0, 0)),
            pl.BlockSpec((64, 128), lambda i: (0, 0)),
            pl.BlockSpec((64, 128), lambda i: (0, 0)),
        ],
        out_specs=[
            pl.BlockSpec((NBLK, 64), lambda i: (i, 0)),
            pl.BlockSpec((NBLK, 128), lambda i: (i, 0)),
            pl.BlockSpec((NBLK, 128), lambda i: (i, 0)),
        ],
        out_shape=[
            jax.ShapeDtypeStruct((NP_, 64), _f32),
            jax.ShapeDtypeStruct((NP_, 128), _f32),
            jax.ShapeDtypeStruct((NP_, 128), _f32),
        ],
    )(zc3, xs_p, ae_pad, waet, b0, wxs, wdt, wst)


def _efs_body(ea_ref, w_ref, o_ref):
    o_ref[...] = jnp.dot(ea_ref[...], w_ref[...], preferred_element_type=_f32, precision=lax.Precision.HIGHEST)


def _efs_call(ea_aug, w_aug):
    return pl.pallas_call(
        _efs_body,
        grid=(EGRID,),
        in_specs=[
            pl.BlockSpec((EBLK, 24), lambda i: (i, 0)),
            pl.BlockSpec((24, 128), lambda i: (0, 0)),
        ],
        out_specs=pl.BlockSpec((EBLK, 128), lambda i: (i, 0)),
        out_shape=jax.ShapeDtypeStruct((E, 128), _f32),
    )(ea_aug, w_aug)


def _ln_relu(x, agg, g_ref, b_ref):
    y = x + agg
    mu = jnp.mean(y, axis=1, keepdims=True)
    d = y - mu
    var = jnp.mean(d * d, axis=1, keepdims=True)
    xn = d * lax.rsqrt(var + 1e-5) * g_ref[...] + b_ref[...]
    return jnp.maximum(xn, 0.0)


def _post_xform_body(x_ref, agg_ref, g_ref, b_ref, wdt_ref, wst_ref,
                     xn_ref, afs_ref, bfs_ref):
    xn = _ln_relu(x_ref[...], agg_ref[...], g_ref, b_ref)
    xn_ref[...] = xn
    afs_ref[...] = jnp.dot(xn, wdt_ref[...], preferred_element_type=_f32, precision=lax.Precision.HIGHEST)
    bfs_ref[...] = jnp.dot(xn, wst_ref[...], preferred_element_type=_f32, precision=lax.Precision.HIGHEST)


def _post_xform_call(x, agg, g, b, wdt, wst):
    return pl.pallas_call(
        _post_xform_body,
        grid=(NGRID,),
        in_specs=[
            pl.BlockSpec((NBLK, 64), lambda i: (i, 0)),
            pl.BlockSpec((NBLK, 64), lambda i: (i, 0)),
            pl.BlockSpec((1, 64), lambda i: (0, 0)),
            pl.BlockSpec((1, 64), lambda i: (0, 0)),
            pl.BlockSpec((64, 128), lambda i: (0, 0)),
            pl.BlockSpec((64, 128), lambda i: (0, 0)),
        ],
        out_specs=[
            pl.BlockSpec((NBLK, 64), lambda i: (i, 0)),
            pl.BlockSpec((NBLK, 128), lambda i: (i, 0)),
            pl.BlockSpec((NBLK, 128), lambda i: (i, 0)),
        ],
        out_shape=[
            jax.ShapeDtypeStruct((NP_, 64), _f32),
            jax.ShapeDtypeStruct((NP_, 128), _f32),
            jax.ShapeDtypeStruct((NP_, 128), _f32),
        ],
    )(x, agg, g, b, wdt, wst)


def _post_body(x_ref, agg_ref, g_ref, b_ref, xn_ref):
    xn_ref[...] = _ln_relu(x_ref[...], agg_ref[...], g_ref, b_ref)


def _post_call(x, agg, g, b):
    return pl.pallas_call(
        _post_body,
        grid=(NGRID,),
        in_specs=[
            pl.BlockSpec((NBLK, 64), lambda i: (i, 0)),
            pl.BlockSpec((NBLK, 64), lambda i: (i, 0)),
            pl.BlockSpec((1, 64), lambda i: (0, 0)),
            pl.BlockSpec((1, 64), lambda i: (0, 0)),
        ],
        out_specs=pl.BlockSpec((NBLK, 64), lambda i: (i, 0)),
        out_shape=jax.ShapeDtypeStruct((NP_, 64), _f32),
    )(x, agg, g, b)


def _pool_body(x_ref, b3_ref, w1t_ref, b1_ref, w2t_ref, b2_ref, o_ref, acc_ref):
    i = pl.program_id(0)

    @pl.when(i == 0)
    def _():
        acc_ref[...] = jnp.zeros_like(acc_ref)

    bb = b3_ref[0, 0, :]
    oh = (bb[:, None] == lax.broadcasted_iota(jnp.int32, (NBLK, NG), 1)).astype(_f32)
    xa = jnp.concatenate([x_ref[...], jnp.ones((NBLK, 64), _f32)], axis=1)
    acc_ref[...] += lax.dot_general(oh, xa, (((0,), (0,)), ((), ())),
                                    preferred_element_type=_f32, precision=lax.Precision.HIGHEST)

    @pl.when(i == NGRID - 1)
    def _():
        s = acc_ref[:, :64]
        c = acc_ref[:, 64:65]
        pooled = s / jnp.maximum(c, 1.0)
        h = jnp.maximum(jnp.dot(pooled, w1t_ref[...], preferred_element_type=_f32, precision=lax.Precision.HIGHEST)
                        + b1_ref[...], 0.0)
        o_ref[...] = jnp.dot(h, w2t_ref[...], preferred_element_type=_f32, precision=lax.Precision.HIGHEST) + b2_ref[...]


def _pool_call(x, batch3, w1t, b1, w2t, b2):
    return pl.pallas_call(
        _pool_body,
        grid=(NGRID,),
        in_specs=[
            pl.BlockSpec((NBLK, 64), lambda i: (i, 0)),
            pl.BlockSpec((1, 1, NBLK), lambda i: (i, 0, 0)),
            pl.BlockSpec((64, 32), lambda i: (0, 0)),
            pl.BlockSpec((1, 32), lambda i: (0, 0)),
            pl.BlockSpec((32, 8), lambda i: (0, 0)),
            pl.BlockSpec((1, 8), lambda i: (0, 0)),
        ],
        out_specs=pl.BlockSpec((NG, 8), lambda i: (0, 0)),
        out_shape=jax.ShapeDtypeStruct((NG, 8), _f32),
        scratch_shapes=[pltpu.VMEM((NG, 128), _f32)],
    )(x, batch3, w1t, b1, w2t, b2)


def _edge_stage(afs, bfs, efs, src, dst):
    gp = afs[dst] + bfs[src] + efs
    gate = jax.nn.sigmoid(gp[:, :64])
    core = _softplus(gp[:, 64:])
    return jax.ops.segment_sum(gate * core, dst, num_segments=NP_)


NC_SC = 2
NS_SC = 16
NW = NC_SC * NS_SC
CCH = 224
NCHUNK = NP_ // CCH
CPT = NCHUNK // NW
EB = 128


def _edge_sc(dst_s, src_s, eid, eoff_pad, afs, bfs, efs, zrows):
    mesh = plsc.VectorSubcoreMesh(core_axis_name="c", subcore_axis_name="s")

    @functools.partial(
        pl.kernel,
        out_type=jax.ShapeDtypeStruct((NP_, 64), _f32),
        mesh=mesh,
        compiler_params=pltpu.CompilerParams(needs_layout_passes=False),
        scratch_types=[
            pltpu.VMEM((16, 16), jnp.int32),
            pltpu.VMEM((CCH, 128), _f32),
            pltpu.VMEM((CCH, 64), _f32),
        ]
        + [pltpu.VMEM((EB,), jnp.int32) for _ in range(12)]
        + [pltpu.VMEM((EB, 128), _f32) for _ in range(4)]
        + [pltpu.SemaphoreType.DMA for _ in range(6)],
    )
    def k(dst_hbm, src_hbm, eid_hbm, eoff_hbm, afs_hbm, bfs_hbm, efs_hbm, z_hbm,
          agg_hbm, eoff_v, afsb, accb,
          src0, src1, src2, src3, eid0, eid1, eid2, eid3, dst0, dst1, dst2, dst3,
          bfs0, bfs1, efs0, efs1, semg0, semg1, semi0, semi1, semi2, semi3):
        wid = lax.axis_index("s") * NC_SC + lax.axis_index("c")
        srcs = [src0, src1, src2, src3]
        eids = [eid0, eid1, eid2, eid3]
        dsts = [dst0, dst1, dst2, dst3]
        datab = [(bfs0, efs0), (bfs1, efs1)]
        semg = [semg0, semg1]
        semi = [semi0, semi1, semi2, semi3]
        pltpu.sync_copy(eoff_hbm, eoff_v)
        iota16 = lax.iota(jnp.int32, 16)

        def fire_idx(bi, q):
            e0 = bi * EB
            pltpu.async_copy(src_hbm.at[pl.ds(e0, EB)], srcs[q], semi[q])
            pltpu.async_copy(eid_hbm.at[pl.ds(e0, EB)], eids[q], semi[q])
            pltpu.async_copy(dst_hbm.at[pl.ds(e0, EB)], dsts[q], semi[q])

        def wait_idx(q):
            pltpu.make_async_copy(src_hbm.at[pl.ds(0, EB)], srcs[q], semi[q]).wait()
            pltpu.make_async_copy(src_hbm.at[pl.ds(0, EB)], eids[q], semi[q]).wait()
            pltpu.make_async_copy(src_hbm.at[pl.ds(0, EB)], dsts[q], semi[q]).wait()

        def fire_g(q, p):
            pltpu.async_copy(bfs_hbm.at[srcs[q]], datab[p][0], semg[p])
            pltpu.async_copy(efs_hbm.at[eids[q]], datab[p][1], semg[p])

        def wait_g(q, p):
            pltpu.make_async_copy(bfs_hbm.at[srcs[q]], datab[p][0], semg[p]).wait()
            pltpu.make_async_copy(efs_hbm.at[eids[q]], datab[p][1], semg[p]).wait()

        def compute(cn0, p, q):
            bfsb, efsb = datab[p]
            dstb = dsts[q]

            def group_body(g, carry3):
                ei = iota16 + g * 16
                d16 = plsc.load_gather(dstb, [ei])
                dloc = d16 - cn0
                valid = (dloc >= 0) & (dloc < CCH)
                arow = jnp.clip(dloc, 0, CCH - 1)

                def feat_body(kf, carry4):
                    cf = jnp.broadcast_to(kf, (16,))
                    cs = cf + 64
                    af = plsc.load_gather(afsb, [arow, cf])
                    bf = plsc.load_gather(bfsb, [ei, cf])
                    ef = plsc.load_gather(efsb, [ei, cf])
                    as_ = plsc.load_gather(afsb, [arow, cs])
                    bs = plsc.load_gather(bfsb, [ei, cs])
                    es = plsc.load_gather(efsb, [ei, cs])
                    tf = af + bf + ef
                    ts = as_ + bs + es
                    gate = 1.0 / (1.0 + jnp.exp(-tf))
                    u = jnp.exp(-jnp.abs(ts))
                    zz = u / (2.0 + u)
                    z2 = zz * zz
                    pp = 1.0 + z2 * (1.0 / 3.0 + z2 * (0.2 + z2 * (1.0 / 7.0)))
                    sp = jnp.maximum(ts, 0.0) + 2.0 * zz * pp
                    plsc.addupdate_scatter(accb, [arow, cf], gate * sp, mask=valid)
                    return carry4

                return lax.fori_loop(0, 64, feat_body, carry3, unroll=4)

            lax.fori_loop(0, EB // 16, group_body, 0)

        def chunk_body(j, carry):
            c = wid * CPT + j
            cn0 = c * CCH
            c16 = jnp.full((16,), 0, jnp.int32) + c
            c16b = c16 + 1
            e_lo = jnp.max(plsc.load_gather(eoff_v, [c16 >> 4, c16 & 15]))
            e_hi = jnp.max(plsc.load_gather(eoff_v, [c16b >> 4, c16b & 15]))
            b0 = e_lo // EB
            b1 = (e_hi + (EB - 1)) // EB
            pltpu.sync_copy(z_hbm, accb)
            pltpu.sync_copy(afs_hbm.at[pl.ds(cn0, CCH)], afsb)

            @pl.when(b0 < b1)
            def _():
                e0 = b0 * EB
                pltpu.sync_copy(src_hbm.at[pl.ds(e0, EB)], srcs[0])
                pltpu.sync_copy(eid_hbm.at[pl.ds(e0, EB)], eids[0])
                pltpu.sync_copy(dst_hbm.at[pl.ds(e0, EB)], dsts[0])
                fire_g(0, 0)

                @pl.when(b0 + 1 < b1)
                def _():
                    fire_idx(b0 + 1, 1)

                nquad = (b1 - b0 + 3) // 4

                def quad_body(t, carry2):
                    base = b0 + 4 * t
                    for s in range(4):
                        bi = base + s
                        qs, qn, qn2 = s, (s + 1) % 4, (s + 2) % 4
                        ps, pn = s % 2, (s + 1) % 2

                        @pl.when(bi < b1)
                        def _(bi=bi, qs=qs, qn=qn, qn2=qn2, ps=ps, pn=pn):
                            @pl.when(bi + 1 < b1)
                            def _():
                                wait_idx(qn)
                                fire_g(qn, pn)

                            @pl.when(bi + 2 < b1)
                            def _():
                                fire_idx(bi + 2, qn2)

                            wait_g(qs, ps)
                            compute(cn0, ps, qs)

                    return carry2

                lax.fori_loop(0, nquad, quad_body, 0)

            pltpu.sync_copy(accb, agg_hbm.at[pl.ds(cn0, CCH)])
            return carry

        lax.fori_loop(0, CPT, chunk_body, 0)

    return k(dst_s, src_s, eid, eoff_pad, afs, bfs, efs, zrows)


def kernel(z, x_scalar, edge_index, edge_attr, batch, atom_embed, lin0_w, lin0_b,
           convf_w, convf_b, convs_w, convs_b, ln_g, ln_b, lin1_w, lin1_b,
           lin2_w, lin2_b):
    src = edge_index[0]
    dst = edge_index[1]
    zc = jnp.clip(z, 0, MAX_Z)

    zc3 = jnp.pad(zc, (0, NP_ - N)).reshape(NGRID, 1, NBLK)
    xs_p = jnp.pad(x_scalar, ((0, NP_ - N), (0, 8 - NS)))
    ae_pad = jnp.pad(atom_embed, ((0, 128 - (MAX_Z + 2)), (0, 0)))
    waet = lin0_w[:, :HID].T
    wxs = jnp.pad(lin0_w[:, HID:].T, ((0, 8 - NS), (0, 0)))
    b0 = lin0_b[None, :]

    wdt = [jnp.concatenate([convf_w[l][:, :HID].T, convs_w[l][:, :HID].T], axis=1)
           for l in range(L)]
    wst = [jnp.concatenate([convf_w[l][:, HID:2 * HID].T,
                            convs_w[l][:, HID:2 * HID].T], axis=1) for l in range(L)]
    ea_aug = jnp.concatenate(
        [edge_attr, jnp.ones((E, 1), _f32), jnp.zeros((E, 7), _f32)], axis=1)
    we_aug = [jnp.concatenate([
        jnp.concatenate([convf_w[l][:, 2 * HID:].T, convs_w[l][:, 2 * HID:].T], axis=1),
        jnp.concatenate([convf_b[l], convs_b[l]])[None, :],
        jnp.zeros((7, 128), _f32)], axis=0) for l in range(L)]

    dst_s, src_s, eid = lax.sort(
        (dst, src, jnp.arange(E, dtype=jnp.int32)), num_keys=1)
    bounds = jnp.arange(NCHUNK + 1, dtype=jnp.int32) * CCH
    eoff = jnp.searchsorted(dst_s, bounds, side='left').astype(jnp.int32)
    eoff_pad = jnp.pad(eoff, (0, 256 - (NCHUNK + 1)), constant_values=E).reshape(16, 16)
    zrows = jnp.zeros((CCH, 64), _f32)

    x, afs, bfs = _x0_call(zc3, xs_p, ae_pad, waet, b0, wxs, wdt[0], wst[0])
    efs = [_efs_call(ea_aug, we_aug[l]) for l in range(L)]

    for l in range(L):
        agg = _edge_sc(dst_s, src_s, eid, eoff_pad, afs, bfs, efs[l], zrows)
        if l < L - 1:
            x, afs, bfs = _post_xform_call(x, agg, ln_g[l][None, :], ln_b[l][None, :],
                                           wdt[l + 1], wst[l + 1])
        else:
            x = _post_call(x, agg, ln_g[l][None, :], ln_b[l][None, :])

    batch3 = jnp.pad(batch, (0, NP_ - N), constant_values=NG).reshape(NGRID, 1, NBLK)
    w1t = lin1_w.T
    b1 = lin1_b[None, :]
    w2t = jnp.pad(lin2_w.T, ((0, 0), (0, 7)))
    b2 = jnp.pad(lin2_b[None, :], ((0, 0), (0, 7)))
    out2 = _pool_call(x, batch3, w1t, b1, w2t, b2)
    return out2[:, 0]


# speedup vs baseline: 1.6084x; 1.5113x over previous
import functools
import jax
import jax.numpy as jnp
from jax import lax
from jax.experimental import pallas as pl
from jax.experimental.pallas import tpu as pltpu
from jax.experimental.pallas import tpu_sc as plsc

N = 50000
E = 800000
NG = 256
HID = 64
EDGE = 16
L = 3
NS = 4
MAX_Z = 118

NP_ = 50176
NBLK = 1792
NGRID = 28
EBLK = 4000
EGRID = E // EBLK

_f32 = jnp.float32


def _softplus(x):
    u = jnp.exp(-jnp.abs(x))
    zz = u / (2.0 + u)
    z2 = zz * zz
    p = 1.0 + z2 * (1.0 / 3.0 + z2 * (0.2 + z2 * (1.0 / 7.0)))
    return jnp.maximum(x, 0.0) + 2.0 * zz * p


def _x0_body(zc_ref, xs_ref, ae_ref, waet_ref, b0_ref, wxs_ref, wdt_ref, wst_ref,
             x0_ref, afs_ref, bfs_ref):
    zc = zc_ref[0, 0, :]
    t = jnp.dot(ae_ref[...], waet_ref[...], preferred_element_type=_f32, precision=lax.Precision.HIGHEST)
    oh = (zc[:, None] == lax.broadcasted_iota(jnp.int32, (NBLK, 128), 1)).astype(_f32)
    g = jnp.dot(oh, t, preferred_element_type=_f32, precision=lax.Precision.HIGHEST)
    x0 = g + jnp.dot(xs_ref[...], wxs_ref[...], preferred_element_type=_f32, precision=lax.Precision.HIGHEST) + b0_ref[...]
    x0_ref[...] = x0
    afs_ref[...] = jnp.dot(x0, wdt_ref[...], preferred_element_type=_f32, precision=lax.Precision.HIGHEST)
    bfs_ref[...] = jnp.dot(x0, wst_ref[...], preferred_element_type=_f32, precision=lax.Precision.HIGHEST)


def _x0_call(zc3, xs_p, ae_pad, waet, b0, wxs, wdt, wst):
    return pl.pallas_call(
        _x0_body,
        grid=(NGRID,),
        in_specs=[
            pl.BlockSpec((1, 1, NBLK), lambda i: (i, 0, 0)),
            pl.BlockSpec((NBLK, 8), lambda i: (i, 0)),
            pl.BlockSpec((128, 64), lambda i: (0, 0)),
            pl.BlockSpec((64, 64), lambda i: (0, 0)),
            pl.BlockSpec((1, 64), lambda i: (0, 0)),
            pl.BlockSpec((8, 64), lambda i: (0, 0)),
            pl.BlockSpec((64, 128), lambda i: (0, 0)),
            pl.BlockSpec((64, 128), lambda i: (0, 0)),
        ],
        out_specs=[
            pl.BlockSpec((NBLK, 64), lambda i: (i, 0)),
            pl.BlockSpec((NBLK, 128), lambda i: (i, 0)),
            pl.BlockSpec((NBLK, 128), lambda i: (i, 0)),
        ],
        out_shape=[
            jax.ShapeDtypeStruct((NP_, 64), _f32),
            jax.ShapeDtypeStruct((NP_, 128), _f32),
            jax.ShapeDtypeStruct((NP_, 128), _f32),
        ],
    )(zc3, xs_p, ae_pad, waet, b0, wxs, wdt, wst)


def _efs_body(ea_ref, w_ref, o_ref):
    o_ref[...] = jnp.dot(ea_ref[...], w_ref[...], preferred_element_type=_f32, precision=lax.Precision.HIGHEST)


def _efs_call(ea_aug, w_aug):
    return pl.pallas_call(
        _efs_body,
        grid=(EGRID,),
        in_specs=[
            pl.BlockSpec((EBLK, 24), lambda i: (i, 0)),
            pl.BlockSpec((24, 128), lambda i: (0, 0)),
        ],
        out_specs=pl.BlockSpec((EBLK, 128), lambda i: (i, 0)),
        out_shape=jax.ShapeDtypeStruct((E, 128), _f32),
    )(ea_aug, w_aug)


def _ln_relu(x, agg, g_ref, b_ref):
    y = x + agg
    mu = jnp.mean(y, axis=1, keepdims=True)
    d = y - mu
    var = jnp.mean(d * d, axis=1, keepdims=True)
    xn = d * lax.rsqrt(var + 1e-5) * g_ref[...] + b_ref[...]
    return jnp.maximum(xn, 0.0)


def _post_xform_body(x_ref, agg_ref, g_ref, b_ref, wdt_ref, wst_ref,
                     xn_ref, afs_ref, bfs_ref):
    xn = _ln_relu(x_ref[...], agg_ref[...], g_ref, b_ref)
    xn_ref[...] = xn
    afs_ref[...] = jnp.dot(xn, wdt_ref[...], preferred_element_type=_f32, precision=lax.Precision.HIGHEST)
    bfs_ref[...] = jnp.dot(xn, wst_ref[...], preferred_element_type=_f32, precision=lax.Precision.HIGHEST)


def _post_xform_call(x, agg, g, b, wdt, wst):
    return pl.pallas_call(
        _post_xform_body,
        grid=(NGRID,),
        in_specs=[
            pl.BlockSpec((NBLK, 64), lambda i: (i, 0)),
            pl.BlockSpec((NBLK, 64), lambda i: (i, 0)),
            pl.BlockSpec((1, 64), lambda i: (0, 0)),
            pl.BlockSpec((1, 64), lambda i: (0, 0)),
            pl.BlockSpec((64, 128), lambda i: (0, 0)),
            pl.BlockSpec((64, 128), lambda i: (0, 0)),
        ],
        out_specs=[
            pl.BlockSpec((NBLK, 64), lambda i: (i, 0)),
            pl.BlockSpec((NBLK, 128), lambda i: (i, 0)),
            pl.BlockSpec((NBLK, 128), lambda i: (i, 0)),
        ],
        out_shape=[
            jax.ShapeDtypeStruct((NP_, 64), _f32),
            jax.ShapeDtypeStruct((NP_, 128), _f32),
            jax.ShapeDtypeStruct((NP_, 128), _f32),
        ],
    )(x, agg, g, b, wdt, wst)


def _post_body(x_ref, agg_ref, g_ref, b_ref, xn_ref):
    xn_ref[...] = _ln_relu(x_ref[...], agg_ref[...], g_ref, b_ref)


def _post_call(x, agg, g, b):
    return pl.pallas_call(
        _post_body,
        grid=(NGRID,),
        in_specs=[
            pl.BlockSpec((NBLK, 64), lambda i: (i, 0)),
            pl.BlockSpec((NBLK, 64), lambda i: (i, 0)),
            pl.BlockSpec((1, 64), lambda i: (0, 0)),
            pl.BlockSpec((1, 64), lambda i: (0, 0)),
        ],
        out_specs=pl.BlockSpec((NBLK, 64), lambda i: (i, 0)),
        out_shape=jax.ShapeDtypeStruct((NP_, 64), _f32),
    )(x, agg, g, b)


def _pool_body(x_ref, b3_ref, w1t_ref, b1_ref, w2t_ref, b2_ref, o_ref, acc_ref):
    i = pl.program_id(0)

    @pl.when(i == 0)
    def _():
        acc_ref[...] = jnp.zeros_like(acc_ref)

    bb = b3_ref[0, 0, :]
    oh = (bb[:, None] == lax.broadcasted_iota(jnp.int32, (NBLK, NG), 1)).astype(_f32)
    xa = jnp.concatenate([x_ref[...], jnp.ones((NBLK, 64), _f32)], axis=1)
    acc_ref[...] += lax.dot_general(oh, xa, (((0,), (0,)), ((), ())),
                                    preferred_element_type=_f32, precision=lax.Precision.HIGHEST)

    @pl.when(i == NGRID - 1)
    def _():
        s = acc_ref[:, :64]
        c = acc_ref[:, 64:65]
        pooled = s / jnp.maximum(c, 1.0)
        h = jnp.maximum(jnp.dot(pooled, w1t_ref[...], preferred_element_type=_f32, precision=lax.Precision.HIGHEST)
                        + b1_ref[...], 0.0)
        o_ref[...] = jnp.dot(h, w2t_ref[...], preferred_element_type=_f32, precision=lax.Precision.HIGHEST) + b2_ref[...]


def _pool_call(x, batch3, w1t, b1, w2t, b2):
    return pl.pallas_call(
        _pool_body,
        grid=(NGRID,),
        in_specs=[
            pl.BlockSpec((NBLK, 64), lambda i: (i, 0)),
            pl.BlockSpec((1, 1, NBLK), lambda i: (i, 0, 0)),
            pl.BlockSpec((64, 32), lambda i: (0, 0)),
            pl.BlockSpec((1, 32), lambda i: (0, 0)),
            pl.BlockSpec((32, 8), lambda i: (0, 0)),
            pl.BlockSpec((1, 8), lambda i: (0, 0)),
        ],
        out_specs=pl.BlockSpec((NG, 8), lambda i: (0, 0)),
        out_shape=jax.ShapeDtypeStruct((NG, 8), _f32),
        scratch_shapes=[pltpu.VMEM((NG, 128), _f32)],
    )(x, batch3, w1t, b1, w2t, b2)


def _edge_stage(afs, bfs, efs, src, dst):
    gp = afs[dst] + bfs[src] + efs
    gate = jax.nn.sigmoid(gp[:, :64])
    core = _softplus(gp[:, 64:])
    return jax.ops.segment_sum(gate * core, dst, num_segments=NP_)


NC_SC = 2
NS_SC = 16
NW = NC_SC * NS_SC
CCH = 224
NCHUNK = NP_ // CCH
CPT = NCHUNK // NW
EB = 128


def _edge_sc(dst_s, src_s, eid, eoff_pad, afs, bfs, efs, zrows):
    mesh = plsc.VectorSubcoreMesh(core_axis_name="c", subcore_axis_name="s")

    @functools.partial(
        pl.kernel,
        out_type=jax.ShapeDtypeStruct((NP_, 64), _f32),
        mesh=mesh,
        compiler_params=pltpu.CompilerParams(needs_layout_passes=False),
        scratch_types=[
            pltpu.VMEM((16, 16), jnp.int32),
            pltpu.VMEM((CCH, 128), _f32),
            pltpu.VMEM((CCH, 64), _f32),
        ]
        + [pltpu.VMEM((EB,), jnp.int32) for _ in range(12)]
        + [pltpu.VMEM((EB, 128), _f32) for _ in range(4)]
        + [pltpu.SemaphoreType.DMA for _ in range(6)],
    )
    def k(dst_hbm, src_hbm, eid_hbm, eoff_hbm, afs_hbm, bfs_hbm, efs_hbm, z_hbm,
          agg_hbm, eoff_v, afsb, accb,
          src0, src1, src2, src3, eid0, eid1, eid2, eid3, dst0, dst1, dst2, dst3,
          bfs0, bfs1, efs0, efs1, semg0, semg1, semi0, semi1, semi2, semi3):
        wid = lax.axis_index("s") * NC_SC + lax.axis_index("c")
        srcs = [src0, src1, src2, src3]
        eids = [eid0, eid1, eid2, eid3]
        dsts = [dst0, dst1, dst2, dst3]
        datab = [(bfs0, efs0), (bfs1, efs1)]
        semg = [semg0, semg1]
        semi = [semi0, semi1, semi2, semi3]
        pltpu.sync_copy(eoff_hbm, eoff_v)
        iota16 = lax.iota(jnp.int32, 16)

        def fire_idx(bi, q):
            e0 = bi * EB
            pltpu.async_copy(src_hbm.at[pl.ds(e0, EB)], srcs[q], semi[q])
            pltpu.async_copy(eid_hbm.at[pl.ds(e0, EB)], eids[q], semi[q])
            pltpu.async_copy(dst_hbm.at[pl.ds(e0, EB)], dsts[q], semi[q])

        def wait_idx(q):
            pltpu.make_async_copy(src_hbm.at[pl.ds(0, EB)], srcs[q], semi[q]).wait()
            pltpu.make_async_copy(src_hbm.at[pl.ds(0, EB)], eids[q], semi[q]).wait()
            pltpu.make_async_copy(src_hbm.at[pl.ds(0, EB)], dsts[q], semi[q]).wait()

        def fire_g(q, p):
            pltpu.async_copy(bfs_hbm.at[srcs[q]], datab[p][0], semg[p])
            pltpu.async_copy(efs_hbm.at[eids[q]], datab[p][1], semg[p])

        def wait_g(q, p):
            pltpu.make_async_copy(bfs_hbm.at[srcs[q]], datab[p][0], semg[p]).wait()
            pltpu.make_async_copy(efs_hbm.at[eids[q]], datab[p][1], semg[p]).wait()

        def compute(cn0, p, q):
            bfsb, efsb = datab[p]
            dstb = dsts[q]

            def group_body(g, carry3):
                ei = iota16 + g * 16
                d16 = plsc.load_gather(dstb, [ei])
                dloc = d16 - cn0
                valid = (dloc >= 0) & (dloc < CCH)
                arow = jnp.clip(dloc, 0, CCH - 1)

                @plsc.parallel_loop(0, 64, unroll=8)
                def feat_body(kf):
                    cf = jnp.broadcast_to(kf, (16,))
                    cs = cf + 64
                    af = plsc.load_gather(afsb, [arow, cf])
                    bf = plsc.load_gather(bfsb, [ei, cf])
                    ef = plsc.load_gather(efsb, [ei, cf])
                    as_ = plsc.load_gather(afsb, [arow, cs])
                    bs = plsc.load_gather(bfsb, [ei, cs])
                    es = plsc.load_gather(efsb, [ei, cs])
                    tf = af + bf + ef
                    ts = as_ + bs + es
                    gate = 1.0 / (1.0 + jnp.exp(-tf))
                    u = jnp.exp(-jnp.abs(ts))
                    zz = u / (2.0 + u)
                    z2 = zz * zz
                    pp = 1.0 + z2 * (1.0 / 3.0 + z2 * (0.2 + z2 * (1.0 / 7.0)))
                    sp = jnp.maximum(ts, 0.0) + 2.0 * zz * pp
                    plsc.addupdate_scatter(accb, [arow, cf], gate * sp, mask=valid)

                return carry3

            lax.fori_loop(0, EB // 16, group_body, 0)

        def chunk_body(j, carry):
            c = wid * CPT + j
            cn0 = c * CCH
            c16 = jnp.full((16,), 0, jnp.int32) + c
            c16b = c16 + 1
            e_lo = jnp.max(plsc.load_gather(eoff_v, [c16 >> 4, c16 & 15]))
            e_hi = jnp.max(plsc.load_gather(eoff_v, [c16b >> 4, c16b & 15]))
            b0 = e_lo // EB
            b1 = (e_hi + (EB - 1)) // EB
            pltpu.sync_copy(z_hbm, accb)
            pltpu.sync_copy(afs_hbm.at[pl.ds(cn0, CCH)], afsb)

            @pl.when(b0 < b1)
            def _():
                e0 = b0 * EB
                pltpu.sync_copy(src_hbm.at[pl.ds(e0, EB)], srcs[0])
                pltpu.sync_copy(eid_hbm.at[pl.ds(e0, EB)], eids[0])
                pltpu.sync_copy(dst_hbm.at[pl.ds(e0, EB)], dsts[0])
                fire_g(0, 0)

                @pl.when(b0 + 1 < b1)
                def _():
                    fire_idx(b0 + 1, 1)

                nquad = (b1 - b0 + 3) // 4

                def quad_body(t, carry2):
                    base = b0 + 4 * t
                    for s in range(4):
                        bi = base + s
                        qs, qn, qn2 = s, (s + 1) % 4, (s + 2) % 4
                        ps, pn = s % 2, (s + 1) % 2

                        @pl.when(bi < b1)
                        def _(bi=bi, qs=qs, qn=qn, qn2=qn2, ps=ps, pn=pn):
                            @pl.when(bi + 1 < b1)
                            def _():
                                wait_idx(qn)
                                fire_g(qn, pn)

                            @pl.when(bi + 2 < b1)
                            def _():
                                fire_idx(bi + 2, qn2)

                            wait_g(qs, ps)
                            compute(cn0, ps, qs)

                    return carry2

                lax.fori_loop(0, nquad, quad_body, 0)

            pltpu.sync_copy(accb, agg_hbm.at[pl.ds(cn0, CCH)])
            return carry

        lax.fori_loop(0, CPT, chunk_body, 0)

    return k(dst_s, src_s, eid, eoff_pad, afs, bfs, efs, zrows)


def kernel(z, x_scalar, edge_index, edge_attr, batch, atom_embed, lin0_w, lin0_b,
           convf_w, convf_b, convs_w, convs_b, ln_g, ln_b, lin1_w, lin1_b,
           lin2_w, lin2_b):
    src = edge_index[0]
    dst = edge_index[1]
    zc = jnp.clip(z, 0, MAX_Z)

    zc3 = jnp.pad(zc, (0, NP_ - N)).reshape(NGRID, 1, NBLK)
    xs_p = jnp.pad(x_scalar, ((0, NP_ - N), (0, 8 - NS)))
    ae_pad = jnp.pad(atom_embed, ((0, 128 - (MAX_Z + 2)), (0, 0)))
    waet = lin0_w[:, :HID].T
    wxs = jnp.pad(lin0_w[:, HID:].T, ((0, 8 - NS), (0, 0)))
    b0 = lin0_b[None, :]

    wdt = [jnp.concatenate([convf_w[l][:, :HID].T, convs_w[l][:, :HID].T], axis=1)
           for l in range(L)]
    wst = [jnp.concatenate([convf_w[l][:, HID:2 * HID].T,
                            convs_w[l][:, HID:2 * HID].T], axis=1) for l in range(L)]
    ea_aug = jnp.concatenate(
        [edge_attr, jnp.ones((E, 1), _f32), jnp.zeros((E, 7), _f32)], axis=1)
    we_aug = [jnp.concatenate([
        jnp.concatenate([convf_w[l][:, 2 * HID:].T, convs_w[l][:, 2 * HID:].T], axis=1),
        jnp.concatenate([convf_b[l], convs_b[l]])[None, :],
        jnp.zeros((7, 128), _f32)], axis=0) for l in range(L)]

    dst_s, src_s, eid = lax.sort(
        (dst, src, jnp.arange(E, dtype=jnp.int32)), num_keys=1)
    bounds = jnp.arange(NCHUNK + 1, dtype=jnp.int32) * CCH
    eoff = jnp.searchsorted(dst_s, bounds, side='left').astype(jnp.int32)
    eoff_pad = jnp.pad(eoff, (0, 256 - (NCHUNK + 1)), constant_values=E).reshape(16, 16)
    zrows = jnp.zeros((CCH, 64), _f32)

    x, afs, bfs = _x0_call(zc3, xs_p, ae_pad, waet, b0, wxs, wdt[0], wst[0])
    efs = [_efs_call(ea_aug, we_aug[l]) for l in range(L)]

    for l in range(L):
        agg = _edge_sc(dst_s, src_s, eid, eoff_pad, afs, bfs, efs[l], zrows)
        if l < L - 1:
            x, afs, bfs = _post_xform_call(x, agg, ln_g[l][None, :], ln_b[l][None, :],
                                           wdt[l + 1], wst[l + 1])
        else:
            x = _post_call(x, agg, ln_g[l][None, :], ln_b[l][None, :])

    batch3 = jnp.pad(batch, (0, NP_ - N), constant_values=NG).reshape(NGRID, 1, NBLK)
    w1t = lin1_w.T
    b1 = lin1_b[None, :]
    w2t = jnp.pad(lin2_w.T, ((0, 0), (0, 7)))
    b2 = jnp.pad(lin2_b[None, :], ((0, 0), (0, 7)))
    out2 = _pool_call(x, batch3, w1t, b1, w2t, b2)
    return out2[:, 0]
